# Initial kernel scaffold; baseline (speedup 1.0000x reference)
import jax
import jax.numpy as jnp
from jax.experimental import pallas as pl


def kernel(x, edge_index, W1, att_src1, att_dst1, b1, W2, att_src2, att_dst2, b2):
    raise NotImplementedError("write your pallas kernel here")


# speedup vs baseline: 50.6791x; 50.6791x over previous
import functools

import jax
import jax.numpy as jnp
import numpy as np
from jax import lax
from jax.experimental import pallas as pl
from jax.experimental.pallas import tpu as pltpu
from jax.experimental.pallas import tpu_sc as plsc

N = 10000
E = 320000
F_IN = 128
HID = 16
HEADS = 8
NUM_CLASSES = 40

D1 = HEADS * HID + 16
D2 = 48
CHUNK = 128
NCHUNKS = E // CHUNK
NWORKERS = 32
N_PAD = 10240
ROWS_PER_TILE = N_PAD // 16

_PERM = np.array([(j % HEADS) * HID + j // HEADS for j in range(HEADS * HID)],
                 dtype=np.int32)
_OH = np.equal(np.arange(HEADS * HID)[:, None] % HEADS,
               np.arange(HEADS)[None, :]).astype(np.float32)


def _leaky(x):
    return jnp.maximum(x, 0.2 * x)


def _tc1_body(x_ref, w_ref, asp_ref, adp_ref, t1_ref, a1d_ref, c1_ref,
              msd_ref):
    i = pl.program_id(0)
    nsteps = pl.num_programs(0)
    h = jnp.dot(x_ref[...], w_ref[...], preferred_element_type=jnp.float32)
    a_s = jnp.dot(h, asp_ref[...], preferred_element_type=jnp.float32)
    a_d = jnp.dot(h, adp_ref[...], preferred_element_type=jnp.float32)
    t1_ref[...] = jnp.concatenate([h, a_s, a_s], axis=1)
    a1d_ref[...] = jnp.concatenate([a_d, a_d], axis=1)
    msd = jnp.concatenate([jnp.max(a_s, axis=0),
                           jnp.max(a_d, axis=0)]).reshape(1, 16)

    @pl.when(i == 0)
    def _():
        msd_ref[...] = msd

    @pl.when(i > 0)
    def _():
        msd_ref[...] = jnp.maximum(msd_ref[...], msd)

    @pl.when(i == nsteps - 1)
    def _():
        s = msd_ref[0, 0:8] + msd_ref[0, 8:16]
        c = _leaky(s)
        c1_ref[...] = jnp.concatenate([c, c]).reshape(1, 16)


def _tc2_body(acc_ref, t1_ref, a1d_ref, c1_ref, b1p_ref, w2p_ref, as2_ref,
              ad2_ref, t2_ref, a2s_ref, a2d_ref, c2_ref, msd_ref):
    i = pl.program_id(0)
    nsteps = pl.num_programs(0)
    accs = acc_ref[0] + acc_ref[1]
    num = accs[:, 0:128]
    den8 = accs[:, 128:136]
    a_s = t1_ref[:, 128:136]
    a_d = a1d_ref[:, 0:8]
    p_self = jnp.exp(_leaky(a_s + a_d) - c1_ref[0:1, 0:8])
    h1 = t1_ref[:, 0:128]
    p_tiled = jnp.concatenate([p_self] * 16, axis=1)
    den = den8 + p_self
    den_t = jnp.concatenate([den] * 16, axis=1)
    out1 = (num + p_tiled * h1) / den_t
    z = out1 + b1p_ref[0:1, :]
    x2 = jnp.where(z > 0, z, jnp.exp(z) - 1.0)
    h2 = jnp.dot(x2, w2p_ref[...], preferred_element_type=jnp.float32)
    a2s = jnp.dot(h2, as2_ref[...], preferred_element_type=jnp.float32)
    a2d = jnp.dot(h2, ad2_ref[...], preferred_element_type=jnp.float32)
    ones = jnp.ones((h2.shape[0], 1), jnp.float32)
    zeros = jnp.zeros((h2.shape[0], 7), jnp.float32)
    t2_ref[...] = jnp.concatenate([h2, ones, zeros], axis=1)
    a2s_ref[...] = jnp.concatenate([a2s] * 16, axis=1)
    a2d_ref[...] = jnp.concatenate([a2d] * 16, axis=1)
    msd = jnp.concatenate([jnp.max(a2s, axis=0),
                           jnp.max(a2d, axis=0)]).reshape(1, 2)

    @pl.when(i == 0)
    def _():
        msd_ref[...] = msd

    @pl.when(i > 0)
    def _():
        msd_ref[...] = jnp.maximum(msd_ref[...], msd)

    @pl.when(i == nsteps - 1)
    def _():
        c2 = _leaky(msd_ref[0, 0:1] + msd_ref[0, 1:2])
        c2_ref[...] = jnp.broadcast_to(c2.reshape(1, 1), (1, 16))


def _tc3_body(acc_ref, t2_ref, a2s_ref, a2d_ref, c2_ref, b2_ref, out_ref):
    accs = acc_ref[0] + acc_ref[1]
    num = accs[:, 0:40]
    den = accs[:, 40:41]
    a_s = a2s_ref[:, 0:1]
    a_d = a2d_ref[:, 0:1]
    p_self = jnp.exp(_leaky(a_s + a_d) - c2_ref[0:1, 0:1])
    h2 = t2_ref[:, 0:40]
    o = (num + p_self * h2) / (den + p_self) + b2_ref[0:1, :]
    m = jnp.max(o, axis=1, keepdims=True)
    lo = o - m
    out_ref[...] = lo - jnp.log(jnp.sum(jnp.exp(lo), axis=1, keepdims=True))


_MESH = plsc.VectorSubcoreMesh(core_axis_name="c", subcore_axis_name="s",
                               num_cores=2, num_subcores=16)


@functools.partial(
    pl.kernel,
    out_type=jax.ShapeDtypeStruct((2, N_PAD, D1), jnp.float32),
    mesh=_MESH,
    compiler_params=pltpu.CompilerParams(use_tc_tiling_on_sc=False),
    scratch_types=[
        pltpu.VMEM((CHUNK,), jnp.int32),
        pltpu.VMEM((CHUNK,), jnp.int32),
        pltpu.VMEM((CHUNK, D1), jnp.float32),
        pltpu.VMEM((CHUNK, 16), jnp.float32),
        pltpu.VMEM((16,), jnp.float32),
        pltpu.VMEM_SHARED((N_PAD, D1), jnp.float32),
        pltpu.SemaphoreType.DMA,
        pltpu.SemaphoreType.DMA,
    ],
)
def _sc_edge_pass1(src_hbm, dst_hbm, t1_hbm, a1d_hbm, c1_hbm, zero_hbm,
                   out_hbm, idx_s, idx_d, rows, adrows, cvec, acc, sem1, sem2):
    cid = lax.axis_index("c")
    sid = lax.axis_index("s")
    w = sid * 2 + cid
    pltpu.sync_copy(c1_hbm, cvec)
    pltpu.sync_copy(zero_hbm.at[pl.ds(sid * ROWS_PER_TILE, ROWS_PER_TILE)],
                    acc.at[pl.ds(sid * ROWS_PER_TILE, ROWS_PER_TILE)])
    plsc.subcore_barrier()

    nch = (NCHUNKS - w + NWORKERS - 1) // NWORKERS

    def chunk_body(i, _):
        ch = w + i * NWORKERS
        pltpu.sync_copy(src_hbm.at[pl.ds(ch * CHUNK, CHUNK)], idx_s)
        pltpu.sync_copy(dst_hbm.at[pl.ds(ch * CHUNK, CHUNK)], idx_d)
        cp1 = pltpu.async_copy(t1_hbm.at[idx_s], rows, sem1)
        cp2 = pltpu.async_copy(a1d_hbm.at[idx_d], adrows, sem2)
        cp1.wait()
        cp2.wait()
        cv = cvec[...]

        def edge_body(k, _):
            a_s = rows[k, pl.ds(128, 16)]
            a_d = adrows[k, :]
            p = jnp.exp(_leaky(a_s + a_d) - cv)
            rows[k, pl.ds(128, 16)] = p
            for j in range(8):
                rows[k, pl.ds(j * 16, 16)] = rows[k, pl.ds(j * 16, 16)] * p
            return 0

        lax.fori_loop(0, CHUNK, edge_body, 0, unroll=2)
        pltpu.sync_copy(rows, acc.at[idx_d], add=True)
        return 0

    lax.fori_loop(0, nch, chunk_body, 0)
    plsc.subcore_barrier()
    pltpu.sync_copy(acc.at[pl.ds(sid * ROWS_PER_TILE, ROWS_PER_TILE)],
                    out_hbm.at[cid, pl.ds(sid * ROWS_PER_TILE, ROWS_PER_TILE)])


@functools.partial(
    pl.kernel,
    out_type=jax.ShapeDtypeStruct((2, N_PAD, D2), jnp.float32),
    mesh=_MESH,
    compiler_params=pltpu.CompilerParams(use_tc_tiling_on_sc=False),
    scratch_types=[
        pltpu.VMEM((CHUNK,), jnp.int32),
        pltpu.VMEM((CHUNK,), jnp.int32),
        pltpu.VMEM((CHUNK, D2), jnp.float32),
        pltpu.VMEM((CHUNK, 16), jnp.float32),
        pltpu.VMEM((CHUNK, 16), jnp.float32),
        pltpu.VMEM((16,), jnp.float32),
        pltpu.VMEM_SHARED((N_PAD, D2), jnp.float32),
        pltpu.SemaphoreType.DMA,
        pltpu.SemaphoreType.DMA,
        pltpu.SemaphoreType.DMA,
    ],
)
def _sc_edge_pass2(src_hbm, dst_hbm, t2_hbm, a2s_hbm, a2d_hbm, c2_hbm,
                   zero_hbm, out_hbm, idx_s, idx_d, rows, asrows, adrows,
                   cvec, acc, sem1, sem2, sem3):
    cid = lax.axis_index("c")
    sid = lax.axis_index("s")
    w = sid * 2 + cid
    pltpu.sync_copy(c2_hbm, cvec)
    pltpu.sync_copy(zero_hbm.at[pl.ds(sid * ROWS_PER_TILE, ROWS_PER_TILE)],
                    acc.at[pl.ds(sid * ROWS_PER_TILE, ROWS_PER_TILE)])
    plsc.subcore_barrier()

    nch = (NCHUNKS - w + NWORKERS - 1) // NWORKERS

    def chunk_body(i, _):
        ch = w + i * NWORKERS
        pltpu.sync_copy(src_hbm.at[pl.ds(ch * CHUNK, CHUNK)], idx_s)
        pltpu.sync_copy(dst_hbm.at[pl.ds(ch * CHUNK, CHUNK)], idx_d)
        cp1 = pltpu.async_copy(t2_hbm.at[idx_s], rows, sem1)
        cp2 = pltpu.async_copy(a2s_hbm.at[idx_s], asrows, sem2)
        cp3 = pltpu.async_copy(a2d_hbm.at[idx_d], adrows, sem3)
        cp1.wait()
        cp2.wait()
        cp3.wait()
        cv = cvec[...]

        def edge_body(k, _):
            a_s = asrows[k, :]
            a_d = adrows[k, :]
            p = jnp.exp(_leaky(a_s + a_d) - cv)
            for j in range(3):
                rows[k, pl.ds(j * 16, 16)] = rows[k, pl.ds(j * 16, 16)] * p
            return 0

        lax.fori_loop(0, CHUNK, edge_body, 0, unroll=2)
        pltpu.sync_copy(rows, acc.at[idx_d], add=True)
        return 0

    lax.fori_loop(0, nch, chunk_body, 0)
    plsc.subcore_barrier()
    pltpu.sync_copy(acc.at[pl.ds(sid * ROWS_PER_TILE, ROWS_PER_TILE)],
                    out_hbm.at[cid, pl.ds(sid * ROWS_PER_TILE, ROWS_PER_TILE)])


def kernel(x, edge_index, W1, att_src1, att_dst1, b1, W2, att_src2, att_dst2,
           b2):
    src1d = edge_index[0]
    dst1d = edge_index[1]

    perm = jnp.asarray(_PERM)
    W1p = W1[:, perm]
    b1p = b1[perm].reshape(1, HEADS * HID)
    W2p = W2[perm, :]
    att_s1p = att_src1.reshape(HEADS * HID)[perm]
    att_d1p = att_dst1.reshape(HEADS * HID)[perm]
    oh = jnp.asarray(_OH)
    Asp = att_s1p[:, None] * oh
    Adp = att_d1p[:, None] * oh
    As2 = att_src2.reshape(NUM_CLASSES, 1)
    Ad2 = att_dst2.reshape(NUM_CLASSES, 1)
    b2r = b2.reshape(1, NUM_CLASSES)

    f32 = jnp.float32
    B = 1000
    nb = N // B
    t1, a1d, c1, _ = pl.pallas_call(
        _tc1_body,
        grid=(nb,),
        in_specs=[
            pl.BlockSpec((B, F_IN), lambda i: (i, 0)),
            pl.BlockSpec((F_IN, F_IN), lambda i: (0, 0)),
            pl.BlockSpec((F_IN, HEADS), lambda i: (0, 0)),
            pl.BlockSpec((F_IN, HEADS), lambda i: (0, 0)),
        ],
        out_specs=[
            pl.BlockSpec((B, D1), lambda i: (i, 0)),
            pl.BlockSpec((B, 16), lambda i: (i, 0)),
            pl.BlockSpec((1, 16), lambda i: (0, 0)),
            pl.BlockSpec((1, 16), lambda i: (0, 0)),
        ],
        out_shape=[
            jax.ShapeDtypeStruct((N, D1), f32),
            jax.ShapeDtypeStruct((N, 16), f32),
            jax.ShapeDtypeStruct((1, 16), f32),
            jax.ShapeDtypeStruct((1, 16), f32),
        ],
    )(x, W1p, Asp, Adp)

    zero1 = jnp.zeros((N_PAD, D1), f32)
    acc1 = _sc_edge_pass1(src1d, dst1d, t1, a1d, c1.reshape(16), zero1)

    acc1 = acc1[:, :N, :]
    t2, a2s, a2d, c2, _ = pl.pallas_call(
        _tc2_body,
        grid=(nb,),
        in_specs=[
            pl.BlockSpec((2, B, D1), lambda i: (0, i, 0)),
            pl.BlockSpec((B, D1), lambda i: (i, 0)),
            pl.BlockSpec((B, 16), lambda i: (i, 0)),
            pl.BlockSpec((1, 16), lambda i: (0, 0)),
            pl.BlockSpec((1, F_IN), lambda i: (0, 0)),
            pl.BlockSpec((F_IN, NUM_CLASSES), lambda i: (0, 0)),
            pl.BlockSpec((NUM_CLASSES, 1), lambda i: (0, 0)),
            pl.BlockSpec((NUM_CLASSES, 1), lambda i: (0, 0)),
        ],
        out_specs=[
            pl.BlockSpec((B, D2), lambda i: (i, 0)),
            pl.BlockSpec((B, 16), lambda i: (i, 0)),
            pl.BlockSpec((B, 16), lambda i: (i, 0)),
            pl.BlockSpec((1, 16), lambda i: (0, 0)),
            pl.BlockSpec((1, 2), lambda i: (0, 0)),
        ],
        out_shape=[
            jax.ShapeDtypeStruct((N, D2), f32),
            jax.ShapeDtypeStruct((N, 16), f32),
            jax.ShapeDtypeStruct((N, 16), f32),
            jax.ShapeDtypeStruct((1, 16), f32),
            jax.ShapeDtypeStruct((1, 2), f32),
        ],
    )(acc1, t1, a1d, c1, b1p, W2p, As2, Ad2)

    zero2 = jnp.zeros((N_PAD, D2), f32)
    acc2 = _sc_edge_pass2(src1d, dst1d, t2, a2s, a2d, c2.reshape(16), zero2)

    acc2 = acc2[:, :N, :]
    out = pl.pallas_call(
        _tc3_body,
        grid=(nb,),
        in_specs=[
            pl.BlockSpec((2, B, D2), lambda i: (0, i, 0)),
            pl.BlockSpec((B, D2), lambda i: (i, 0)),
            pl.BlockSpec((B, 16), lambda i: (i, 0)),
            pl.BlockSpec((B, 16), lambda i: (i, 0)),
            pl.BlockSpec((1, 16), lambda i: (0, 0)),
            pl.BlockSpec((1, NUM_CLASSES), lambda i: (0, 0)),
        ],
        out_specs=pl.BlockSpec((B, NUM_CLASSES), lambda i: (i, 0)),
        out_shape=jax.ShapeDtypeStruct((N, NUM_CLASSES), f32),
    )(acc2, t2, a2s, a2d, c2, b2r)
    return out
